# baseline (speedup 1.0000x reference)
import jax
import jax.numpy as jnp
from jax.experimental import pallas as pl
from jax.experimental.pallas import tpu as pltpu

ALPHA = 4


def kernel(frames):
    C, T, H, W = frames.shape
    n_slow = T // ALPHA
    idx = jnp.linspace(0.0, float(T - 1), n_slow).astype(jnp.int32)
    n_copies = C * n_slow

    def dma_gather(idx_ref, src_ref, out_ref, sems):
        copies = []
        for k in range(n_copies):
            c, j = divmod(k, n_slow)
            cp = pltpu.make_async_copy(
                src_ref.at[c, pl.ds(idx_ref[j], 1)],
                out_ref.at[c, pl.ds(j, 1)],
                sems.at[k],
            )
            cp.start()
            copies.append(cp)
        for cp in copies:
            cp.wait()

    slow = pl.pallas_call(
        dma_gather,
        grid_spec=pltpu.PrefetchScalarGridSpec(
            num_scalar_prefetch=1,
            grid=(),
            in_specs=[pl.BlockSpec(memory_space=pl.ANY)],
            out_specs=pl.BlockSpec(memory_space=pl.ANY),
            scratch_shapes=[pltpu.SemaphoreType.DMA((n_copies,))],
        ),
        out_shape=jax.ShapeDtypeStruct((C, n_slow, H, W), frames.dtype),
    )(idx, frames)

    return (slow, frames)


# speedup vs baseline: 6.5460x; 6.5460x over previous
import jax
import jax.numpy as jnp
from jax.experimental import pallas as pl
from jax.experimental.pallas import tpu as pltpu

ALPHA = 4


def _gather_copy(idx_ref, in_ref, out_ref):
    del idx_ref
    out_ref[...] = in_ref[...]


def kernel(frames):
    C, T, H, W = frames.shape
    n_slow = T // ALPHA
    idx = jnp.linspace(0.0, float(T - 1), n_slow).astype(jnp.int32)

    slow = pl.pallas_call(
        _gather_copy,
        grid_spec=pltpu.PrefetchScalarGridSpec(
            num_scalar_prefetch=1,
            grid=(C, n_slow),
            in_specs=[
                pl.BlockSpec((1, 1, H, W), lambda c, j, idx_ref: (c, idx_ref[j], 0, 0)),
            ],
            out_specs=pl.BlockSpec((1, 1, H, W), lambda c, j, idx_ref: (c, j, 0, 0)),
        ),
        out_shape=jax.ShapeDtypeStruct((C, n_slow, H, W), frames.dtype),
    )(idx, frames)

    return (slow, frames)


# speedup vs baseline: 8.5716x; 1.3094x over previous
import jax
import jax.numpy as jnp
from jax.experimental import pallas as pl
from jax.experimental.pallas import tpu as pltpu

ALPHA = 4


def _gather_copy(idx_ref, in_ref, out_ref):
    del idx_ref
    out_ref[...] = in_ref[...]


def kernel(frames):
    C, T, H, W = frames.shape
    n_slow = T // ALPHA
    idx = jnp.linspace(0.0, float(T - 1), n_slow).astype(jnp.int32)

    slow = pl.pallas_call(
        _gather_copy,
        grid_spec=pltpu.PrefetchScalarGridSpec(
            num_scalar_prefetch=1,
            grid=(n_slow,),
            in_specs=[
                pl.BlockSpec((C, 1, H, W), lambda j, idx_ref: (0, idx_ref[j], 0, 0)),
            ],
            out_specs=pl.BlockSpec((C, 1, H, W), lambda j, idx_ref: (0, j, 0, 0)),
        ),
        out_shape=jax.ShapeDtypeStruct((C, n_slow, H, W), frames.dtype),
    )(idx, frames)

    return (slow, frames)


# speedup vs baseline: 9.3605x; 1.0920x over previous
import jax
import jax.numpy as jnp
from jax.experimental import pallas as pl
from jax.experimental.pallas import tpu as pltpu

ALPHA = 4
_P = 2


def _gather_copy(idx_ref, *refs):
    del idx_ref
    in_refs, out_ref = refs[:-1], refs[-1]
    for p, in_ref in enumerate(in_refs):
        out_ref[:, p : p + 1] = in_ref[...]


def kernel(frames):
    C, T, H, W = frames.shape
    n_slow = T // ALPHA
    idx = jnp.linspace(0.0, float(T - 1), n_slow).astype(jnp.int32)

    def in_map(p):
        return lambda s, idx_ref: (0, idx_ref[s * _P + p], 0, 0)

    slow = pl.pallas_call(
        _gather_copy,
        grid_spec=pltpu.PrefetchScalarGridSpec(
            num_scalar_prefetch=1,
            grid=(n_slow // _P,),
            in_specs=[pl.BlockSpec((C, 1, H, W), in_map(p)) for p in range(_P)],
            out_specs=pl.BlockSpec((C, _P, H, W), lambda s, idx_ref: (0, s, 0, 0)),
        ),
        out_shape=jax.ShapeDtypeStruct((C, n_slow, H, W), frames.dtype),
    )(idx, *([frames] * _P))

    return (slow, frames)


# speedup vs baseline: 9.5402x; 1.0192x over previous
import jax
import jax.numpy as jnp
from jax.experimental import pallas as pl
from jax.experimental.pallas import tpu as pltpu

ALPHA = 4
_P = 4


def _gather_copy(idx_ref, *refs):
    del idx_ref
    in_refs, out_ref = refs[:-1], refs[-1]
    for p, in_ref in enumerate(in_refs):
        out_ref[:, p : p + 1] = in_ref[...]


def kernel(frames):
    C, T, H, W = frames.shape
    n_slow = T // ALPHA
    idx = jnp.linspace(0.0, float(T - 1), n_slow).astype(jnp.int32)

    def in_map(p):
        return lambda s, idx_ref: (0, idx_ref[s * _P + p], 0, 0)

    slow = pl.pallas_call(
        _gather_copy,
        grid_spec=pltpu.PrefetchScalarGridSpec(
            num_scalar_prefetch=1,
            grid=(n_slow // _P,),
            in_specs=[pl.BlockSpec((C, 1, H, W), in_map(p)) for p in range(_P)],
            out_specs=pl.BlockSpec((C, _P, H, W), lambda s, idx_ref: (0, s, 0, 0)),
        ),
        out_shape=jax.ShapeDtypeStruct((C, n_slow, H, W), frames.dtype),
    )(idx, *([frames] * _P))

    return (slow, frames)


# speedup vs baseline: 9.7272x; 1.0196x over previous
import jax
import jax.numpy as jnp
from jax.experimental import pallas as pl
from jax.experimental.pallas import tpu as pltpu

ALPHA = 4
_P = 8


def _gather_copy(idx_ref, *refs):
    del idx_ref
    in_refs, out_ref = refs[:-1], refs[-1]
    for p, in_ref in enumerate(in_refs):
        out_ref[:, p : p + 1] = in_ref[...]


def kernel(frames):
    C, T, H, W = frames.shape
    n_slow = T // ALPHA
    idx = jnp.linspace(0.0, float(T - 1), n_slow).astype(jnp.int32)

    def in_map(p):
        return lambda s, idx_ref: (0, idx_ref[s * _P + p], 0, 0)

    slow = pl.pallas_call(
        _gather_copy,
        grid_spec=pltpu.PrefetchScalarGridSpec(
            num_scalar_prefetch=1,
            grid=(n_slow // _P,),
            in_specs=[pl.BlockSpec((C, 1, H, W), in_map(p)) for p in range(_P)],
            out_specs=pl.BlockSpec((C, _P, H, W), lambda s, idx_ref: (0, s, 0, 0)),
        ),
        out_shape=jax.ShapeDtypeStruct((C, n_slow, H, W), frames.dtype),
    )(idx, *([frames] * _P))

    return (slow, frames)


# speedup vs baseline: 9.7422x; 1.0015x over previous
import jax
import jax.numpy as jnp
from jax.experimental import pallas as pl
from jax.experimental.pallas import tpu as pltpu

ALPHA = 4


def kernel(frames):
    C, T, H, W = frames.shape
    n_slow = T // ALPHA
    idx = jnp.linspace(0.0, float(T - 1), n_slow).astype(jnp.int32)

    def dma_gather(idx_ref, src, out, buf, in_sems, out_sems):
        def in_copy(j):
            return pltpu.make_async_copy(
                src.at[:, idx_ref[j]], buf.at[j], in_sems.at[j]
            )

        def out_copy(j):
            return pltpu.make_async_copy(buf.at[j], out.at[:, j], out_sems.at[j])

        for j in range(n_slow):
            in_copy(j).start()
        for j in range(n_slow):
            in_copy(j).wait()
            out_copy(j).start()
        for j in range(n_slow):
            out_copy(j).wait()

    slow = pl.pallas_call(
        dma_gather,
        grid_spec=pltpu.PrefetchScalarGridSpec(
            num_scalar_prefetch=1,
            grid=(),
            in_specs=[pl.BlockSpec(memory_space=pl.ANY)],
            out_specs=pl.BlockSpec(memory_space=pl.ANY),
            scratch_shapes=[
                pltpu.VMEM((n_slow, C, H, W), frames.dtype),
                pltpu.SemaphoreType.DMA((n_slow,)),
                pltpu.SemaphoreType.DMA((n_slow,)),
            ],
        ),
        out_shape=jax.ShapeDtypeStruct((C, n_slow, H, W), frames.dtype),
    )(idx, frames)

    return (slow, frames)
